# baseline (speedup 1.0000x reference)
import functools

import jax
import jax.numpy as jnp
from jax import lax
from jax.experimental import pallas as pl
from jax.experimental.pallas import tpu as pltpu
from jax.experimental.pallas import tpu_sc as plsc

B = 2
NSUB = 16
NCORE = 2
SHIFT = 14

SCALES = [
    (128, 128, 128, 200000, 2048, 128, 512),
    (256, 64, 64, 100000, 1024, 128, 512),
    (512, 32, 32, 50000, 512, 64, 256),
]
ZR = 16


def _make_sc_scatter(C, H, W, N, CHUNK_ROWS, KB, BLK):
    HW = H * W
    P = B * HW
    CF = C // 128
    NCHUNK = P // CHUNK_ROWS
    PCC = NCHUNK // NCORE
    SH = ((N + NSUB - 1) // NSUB + BLK - 1) // BLK * BLK
    NPAD = SH * NSUB
    ACC_ROWS = (CHUNK_ROWS * 3 + 8) * CF
    DUMMY = CHUNK_ROWS * 3
    STRIPE = (CHUNK_ROWS * 3 // NSUB) * CF
    NZ = STRIPE // ZR
    assert DUMMY < (1 << SHIFT) and (NPAD << SHIFT) < (1 << 32)

    mesh = plsc.VectorSubcoreMesh(core_axis_name="c", subcore_axis_name="s")

    @functools.partial(
        pl.kernel,
        mesh=mesh,
        out_type=jax.ShapeDtypeStruct((P * 3 * CF, 128), jnp.float32),
        compiler_params=pltpu.CompilerParams(needs_layout_passes=False),
        scratch_types=[
            pltpu.VMEM((SH,), jnp.int32),
            pltpu.VMEM((SH + KB,), jnp.int32),
            pltpu.VMEM((BLK,), jnp.int32),
            pltpu.VMEM((BLK,), jnp.int32),
            pltpu.VMEM((BLK,), jnp.int32),
            pltpu.VMEM((BLK,), jnp.int32),
            pltpu.VMEM((CF * KB, 128), jnp.float32),
            pltpu.VMEM((CF, KB), jnp.int32),
            pltpu.VMEM((CF, KB), jnp.int32),
            pltpu.VMEM((ZR, 128), jnp.float32),
            pltpu.VMEM_SHARED((ACC_ROWS, 128), jnp.float32),
        ],
    )
    def sc_scatter(b_hbm, t_hbm, y_hbm, x_hbm, feat_hbm, g_hbm,
                   kb, pk, bblk, tblk, yblk, xblk, rows, gstage, lstage, zbuf, acc):
        c = lax.axis_index("c")
        s = lax.axis_index("s")
        base = s * SH
        iota = lax.iota(jnp.int32, 16)
        one = jnp.full((16,), 1, jnp.int32)
        zero = jnp.zeros((16,), jnp.int32)

        def key_block(blk, carry):
            boff = blk * BLK
            pltpu.sync_copy(b_hbm.at[pl.ds(base + boff, BLK)], bblk)
            pltpu.sync_copy(t_hbm.at[pl.ds(base + boff, BLK)], tblk)
            pltpu.sync_copy(y_hbm.at[pl.ds(base + boff, BLK)], yblk)
            pltpu.sync_copy(x_hbm.at[pl.ds(base + boff, BLK)], xblk)

            def key_step(v, carry2):
                off = v * 16
                bv = bblk[pl.ds(off, 16)]
                tv = tblk[pl.ds(off, 16)]
                yv = yblk[pl.ds(off, 16)]
                xv = xblk[pl.ds(off, 16)]
                grp = jnp.where(tv >= 5, one, zero) + jnp.where(tv >= 10, one, zero)
                kb[pl.ds(boff + off, 16)] = (bv * HW + yv * W + xv) * 3 + grp
                return carry2

            lax.fori_loop(0, BLK // 16, key_step, jnp.int32(0))
            return carry

        lax.fori_loop(0, SH // BLK, key_block, jnp.int32(0))

        zv = jnp.zeros((16,), jnp.float32)
        for r in range(ZR):
            for c16 in range(128 // 16):
                zbuf[r, pl.ds(c16 * 16, 16)] = zv

        for j in range(PCC):
            lo3 = (c * PCC + j) * (CHUNK_ROWS * 3)
            hi3 = lo3 + CHUNK_ROWS * 3

            for z in range(NZ):
                pltpu.sync_copy(zbuf, acc.at[pl.ds(s * STRIPE + z * ZR, ZR)])
            plsc.subcore_barrier()

            def scan_step(v, fill):
                off = v * 16
                kv = kb[pl.ds(off, 16)]
                m = (kv >= lo3) & (kv < hi3)
                pkv = ((base + off + iota) << SHIFT) | (kv - lo3)
                plsc.store_compressed(pk.at[pl.ds(fill, 16)], pkv, mask=m)
                return fill + plsc.all_reduce_population_count(m)[0]

            total = lax.fori_loop(0, SH // 16, scan_step, jnp.int32(0))

            for u in range(KB // 16):
                pk[pl.ds(total + u * 16, 16)] = jnp.full((16,), DUMMY, jnp.int32)

            nb = (total + KB - 1) // KB

            def gs_step(v, carry):
                off = v * KB
                for u in range(KB // 16):
                    pkv = pk[pl.ds(off + u * 16, 16)]
                    gv = lax.shift_right_logical(pkv, SHIFT)
                    lv = pkv & ((1 << SHIFT) - 1)
                    for cc in range(CF):
                        gstage[cc, pl.ds(u * 16, 16)] = gv * CF + cc
                        lstage[cc, pl.ds(u * 16, 16)] = lv * CF + cc
                for cc in range(CF):
                    pltpu.sync_copy(feat_hbm.at[gstage.at[cc]],
                                    rows.at[pl.ds(cc * KB, KB)])
                for cc in range(CF):
                    pltpu.sync_copy(rows.at[pl.ds(cc * KB, KB)],
                                    acc.at[lstage.at[cc]], add=True)
                return carry

            lax.fori_loop(0, nb, gs_step, jnp.int32(0))
            plsc.subcore_barrier()

            pltpu.sync_copy(acc.at[pl.ds(s * STRIPE, STRIPE)],
                            g_hbm.at[pl.ds(lo3 * CF + s * STRIPE, STRIPE)])

    return sc_scatter, NPAD, P


def _make_tc_matmul(HW, K3, O, R):
    grid = (B, HW // R)

    def body(g_ref, w_ref, bias_ref, o_ref):
        o_ref[0] = lax.dot_general(
            w_ref[...], g_ref[0],
            dimension_numbers=(((1,), (1,)), ((), ())),
            preferred_element_type=jnp.float32,
            precision=lax.Precision.HIGHEST,
        ) + bias_ref[...]

    return pl.pallas_call(
        body,
        grid=grid,
        in_specs=[
            pl.BlockSpec((1, R, K3), lambda b, r: (b, r, 0)),
            pl.BlockSpec((O, K3), lambda b, r: (0, 0)),
            pl.BlockSpec((O, 1), lambda b, r: (0, 0)),
        ],
        out_specs=pl.BlockSpec((1, O, R), lambda b, r: (b, 0, r)),
        out_shape=jax.ShapeDtypeStruct((B, O, HW), jnp.float32),
    )


_SC_KERNELS = []
_TC_KERNELS = []
for _C, _H, _W, _N, _CR, _KB, _BLK in SCALES:
    _SC_KERNELS.append(_make_sc_scatter(_C, _H, _W, _N, _CR, _KB, _BLK))
    _TC_KERNELS.append(_make_tc_matmul(_H * _W, 3 * _C, _C, min(_H * _W, 2048)))


def _one_scale(i, feat, idx, Wm, bvec):
    C, H, W, N, _, _, _ = SCALES[i]
    sc_scatter, NPAD, P = _SC_KERNELS[i]
    pad = NPAD - N
    bcol = jnp.concatenate([idx[:, 0], jnp.full((pad,), B, jnp.int32)])
    tcol = jnp.concatenate([idx[:, 1], jnp.zeros((pad,), jnp.int32)])
    ycol = jnp.concatenate([idx[:, 2], jnp.zeros((pad,), jnp.int32)])
    xcol = jnp.concatenate([idx[:, 3], jnp.zeros((pad,), jnp.int32)])
    G = sc_scatter(bcol, tcol, ycol, xcol, feat.reshape(N * (C // 128), 128))
    G3 = G.reshape(B, H * W, 3 * C)
    out = _TC_KERNELS[i](G3, Wm, bvec[:, None])
    return out.reshape(B, C, H, W)


def kernel(feat0, idx0, feat1, idx1, feat2, idx2, W0, b0, W1, b1, W2, b2,
           batch_size):
    o0 = _one_scale(0, feat0, idx0, W0, b0)
    o1 = _one_scale(1, feat1, idx1, W1, b1)
    o2 = _one_scale(2, feat2, idx2, W2, b2)
    return (o0, o1, o2)


# speedup vs baseline: 1.4710x; 1.4710x over previous
import functools

import jax
import jax.numpy as jnp
from jax import lax
from jax.experimental import pallas as pl
from jax.experimental.pallas import tpu as pltpu
from jax.experimental.pallas import tpu_sc as plsc

B = 2
NSUB = 16
NCORE = 2
SHIFT = 14

SCALES = [
    (128, 128, 128, 200000, 2048, 64, 512),
    (256, 64, 64, 100000, 1024, 64, 512),
    (512, 32, 32, 50000, 512, 64, 256),
]
ZR = 16


def _make_sc_scatter(C, H, W, N, CHUNK_ROWS, KB, BLK):
    HW = H * W
    P = B * HW
    CF = C // 128
    NCHUNK = P // CHUNK_ROWS
    PCC = NCHUNK // NCORE
    SH = ((N + NSUB - 1) // NSUB + BLK - 1) // BLK * BLK
    NPAD = SH * NSUB
    ACC_ROWS = (CHUNK_ROWS * 3 + 8) * CF
    DUMMY = CHUNK_ROWS * 3
    STRIPE = (CHUNK_ROWS * 3 // NSUB) * CF
    NZ = STRIPE // ZR
    assert DUMMY < (1 << SHIFT) and (NPAD << SHIFT) < (1 << 32)

    mesh = plsc.VectorSubcoreMesh(core_axis_name="c", subcore_axis_name="s")

    @functools.partial(
        pl.kernel,
        mesh=mesh,
        out_type=jax.ShapeDtypeStruct((P * 3 * CF, 128), jnp.float32),
        compiler_params=pltpu.CompilerParams(needs_layout_passes=False),
        scratch_types=[
            pltpu.VMEM((SH,), jnp.int32),
            pltpu.VMEM((SH + KB,), jnp.int32),
            pltpu.VMEM((BLK,), jnp.int32),
            pltpu.VMEM((BLK,), jnp.int32),
            pltpu.VMEM((BLK,), jnp.int32),
            pltpu.VMEM((BLK,), jnp.int32),
            pltpu.VMEM((CF * KB, 128), jnp.float32),
            pltpu.VMEM((CF, KB), jnp.int32),
            pltpu.VMEM((CF, KB), jnp.int32),
            pltpu.VMEM((ZR, 128), jnp.float32),
            pltpu.VMEM_SHARED((ACC_ROWS, 128), jnp.float32),
        ],
    )
    def sc_scatter(b_hbm, t_hbm, y_hbm, x_hbm, feat_hbm, g_hbm,
                   kb, pk, bblk, tblk, yblk, xblk, rows, gstage, lstage, zbuf, acc):
        c = lax.axis_index("c")
        s = lax.axis_index("s")
        base = s * SH
        iota = lax.iota(jnp.int32, 16)
        one = jnp.full((16,), 1, jnp.int32)
        zero = jnp.zeros((16,), jnp.int32)

        def key_block(blk, carry):
            boff = blk * BLK
            pltpu.sync_copy(b_hbm.at[pl.ds(base + boff, BLK)], bblk)
            pltpu.sync_copy(t_hbm.at[pl.ds(base + boff, BLK)], tblk)
            pltpu.sync_copy(y_hbm.at[pl.ds(base + boff, BLK)], yblk)
            pltpu.sync_copy(x_hbm.at[pl.ds(base + boff, BLK)], xblk)

            def key_step(v, carry2):
                off = v * 16
                bv = bblk[pl.ds(off, 16)]
                tv = tblk[pl.ds(off, 16)]
                yv = yblk[pl.ds(off, 16)]
                xv = xblk[pl.ds(off, 16)]
                grp = jnp.where(tv >= 5, one, zero) + jnp.where(tv >= 10, one, zero)
                kb[pl.ds(boff + off, 16)] = (bv * HW + yv * W + xv) * 3 + grp
                return carry2

            lax.fori_loop(0, BLK // 16, key_step, jnp.int32(0))
            return carry

        lax.fori_loop(0, SH // BLK, key_block, jnp.int32(0))

        zv = jnp.zeros((16,), jnp.float32)
        for r in range(ZR):
            for c16 in range(128 // 16):
                zbuf[r, pl.ds(c16 * 16, 16)] = zv

        for j in range(PCC):
            lo3 = (c * PCC + j) * (CHUNK_ROWS * 3)
            hi3 = lo3 + CHUNK_ROWS * 3

            for z in range(NZ):
                pltpu.sync_copy(zbuf, acc.at[pl.ds(s * STRIPE + z * ZR, ZR)])
            plsc.subcore_barrier()

            def scan_step(v, fill):
                off = v * 16
                kv = kb[pl.ds(off, 16)]
                m = (kv >= lo3) & (kv < hi3)
                pkv = ((base + off + iota) << SHIFT) | (kv - lo3)
                plsc.store_compressed(pk.at[pl.ds(fill, 16)], pkv, mask=m)
                return fill + plsc.all_reduce_population_count(m)[0]

            total = lax.fori_loop(0, SH // 16, scan_step, jnp.int32(0))

            for u in range(KB // 16):
                pk[pl.ds(total + u * 16, 16)] = jnp.full((16,), DUMMY, jnp.int32)

            nb = (total + KB - 1) // KB

            def gs_step(v, carry):
                off = v * KB
                for u in range(KB // 16):
                    pkv = pk[pl.ds(off + u * 16, 16)]
                    gv = lax.shift_right_logical(pkv, SHIFT)
                    lv = pkv & ((1 << SHIFT) - 1)
                    for cc in range(CF):
                        gstage[cc, pl.ds(u * 16, 16)] = gv * CF + cc
                        lstage[cc, pl.ds(u * 16, 16)] = lv * CF + cc
                for cc in range(CF):
                    pltpu.sync_copy(feat_hbm.at[gstage.at[cc]],
                                    rows.at[pl.ds(cc * KB, KB)])
                for cc in range(CF):
                    pltpu.sync_copy(rows.at[pl.ds(cc * KB, KB)],
                                    acc.at[lstage.at[cc]], add=True)
                return carry

            lax.fori_loop(0, nb, gs_step, jnp.int32(0))
            plsc.subcore_barrier()

            pltpu.sync_copy(acc.at[pl.ds(s * STRIPE, STRIPE)],
                            g_hbm.at[pl.ds(lo3 * CF + s * STRIPE, STRIPE)])

    return sc_scatter, NPAD, P


def _make_tc_matmul(HW, K3, O, R):
    grid = (B, HW // R)

    def body(g_ref, w_ref, bias_ref, o_ref):
        o_ref[0] = lax.dot_general(
            w_ref[...], g_ref[0],
            dimension_numbers=(((1,), (1,)), ((), ())),
            preferred_element_type=jnp.float32,
            precision=lax.Precision.HIGHEST,
        ) + bias_ref[...]

    return pl.pallas_call(
        body,
        grid=grid,
        in_specs=[
            pl.BlockSpec((1, R, K3), lambda b, r: (b, r, 0)),
            pl.BlockSpec((O, K3), lambda b, r: (0, 0)),
            pl.BlockSpec((O, 1), lambda b, r: (0, 0)),
        ],
        out_specs=pl.BlockSpec((1, O, R), lambda b, r: (b, 0, r)),
        out_shape=jax.ShapeDtypeStruct((B, O, HW), jnp.float32),
    )


_SC_KERNELS = []
_TC_KERNELS = []
for _C, _H, _W, _N, _CR, _KB, _BLK in SCALES:
    _SC_KERNELS.append(_make_sc_scatter(_C, _H, _W, _N, _CR, _KB, _BLK))
    _TC_KERNELS.append(_make_tc_matmul(_H * _W, 3 * _C, _C, min(_H * _W, 2048)))


def _one_scale(i, feat, idx, Wm, bvec):
    C, H, W, N, _, _, _ = SCALES[i]
    sc_scatter, NPAD, P = _SC_KERNELS[i]
    pad = NPAD - N
    bcol = jnp.concatenate([idx[:, 0], jnp.full((pad,), B, jnp.int32)])
    tcol = jnp.concatenate([idx[:, 1], jnp.zeros((pad,), jnp.int32)])
    ycol = jnp.concatenate([idx[:, 2], jnp.zeros((pad,), jnp.int32)])
    xcol = jnp.concatenate([idx[:, 3], jnp.zeros((pad,), jnp.int32)])
    G = sc_scatter(bcol, tcol, ycol, xcol, feat.reshape(N * (C // 128), 128))
    G3 = G.reshape(B, H * W, 3 * C)
    out = _TC_KERNELS[i](G3, Wm, bvec[:, None])
    return out.reshape(B, C, H, W)


def kernel(feat0, idx0, feat1, idx1, feat2, idx2, W0, b0, W1, b1, W2, b2,
           batch_size):
    o0 = _one_scale(0, feat0, idx0, W0, b0)
    o1 = _one_scale(1, feat1, idx1, W1, b1)
    o2 = _one_scale(2, feat2, idx2, W2, b2)
    return (o0, o1, o2)


# speedup vs baseline: 1.7988x; 1.2228x over previous
import functools

import jax
import jax.numpy as jnp
from jax import lax
from jax.experimental import pallas as pl
from jax.experimental.pallas import tpu as pltpu
from jax.experimental.pallas import tpu_sc as plsc

B = 2
NSUB = 16
NCORE = 2
SHIFT = 14

SCALES = [
    (128, 128, 128, 200000, 2048, 64, 512),
    (256, 64, 64, 100000, 1024, 64, 512),
    (512, 32, 32, 50000, 512, 64, 256),
]
ZR = 16


def _make_sc_scatter(C, H, W, N, CHUNK_ROWS, KB, BLK):
    HW = H * W
    P = B * HW
    CF = C // 128
    NCHUNK = P // CHUNK_ROWS
    PCC = NCHUNK // NCORE
    SH = ((N + NSUB - 1) // NSUB + BLK - 1) // BLK * BLK
    NPAD = SH * NSUB
    ACC_ROWS = (CHUNK_ROWS * 3 + 8) * CF
    DUMMY = CHUNK_ROWS * 3
    STRIPE = (CHUNK_ROWS * 3 // NSUB) * CF
    NZ = STRIPE // ZR
    assert DUMMY < (1 << SHIFT) and (NPAD << SHIFT) < (1 << 32)

    mesh = plsc.VectorSubcoreMesh(core_axis_name="c", subcore_axis_name="s")

    @functools.partial(
        pl.kernel,
        mesh=mesh,
        out_type=jax.ShapeDtypeStruct((P * 3 * CF, 128), jnp.float32),
        compiler_params=pltpu.CompilerParams(needs_layout_passes=False),
        scratch_types=[
            pltpu.VMEM((SH,), jnp.int32),
            pltpu.VMEM((SH + KB,), jnp.int32),
            pltpu.VMEM((BLK,), jnp.int32),
            pltpu.VMEM((BLK,), jnp.int32),
            pltpu.VMEM((BLK,), jnp.int32),
            pltpu.VMEM((BLK,), jnp.int32),
            pltpu.VMEM((CF * KB, 128), jnp.float32),
            pltpu.VMEM((CF * KB, 128), jnp.float32),
            pltpu.VMEM((CF, KB), jnp.int32),
            pltpu.VMEM((CF, KB), jnp.int32),
            pltpu.VMEM((CF, KB), jnp.int32),
            pltpu.VMEM((CF, KB), jnp.int32),
            pltpu.VMEM((ZR, 128), jnp.float32),
            pltpu.VMEM_SHARED((ACC_ROWS, 128), jnp.float32),
            pltpu.SemaphoreType.DMA,
            pltpu.SemaphoreType.DMA,
        ],
    )
    def sc_scatter(b_hbm, t_hbm, y_hbm, x_hbm, feat_hbm, g_hbm,
                   kb, pk, bblk, tblk, yblk, xblk, rowsA, rowsB, gstA, gstB,
                   lstA, lstB, zbuf, acc, semA, semB):
        c = lax.axis_index("c")
        s = lax.axis_index("s")
        base = s * SH
        iota = lax.iota(jnp.int32, 16)
        one = jnp.full((16,), 1, jnp.int32)
        zero = jnp.zeros((16,), jnp.int32)

        def key_block(blk, carry):
            boff = blk * BLK
            pltpu.sync_copy(b_hbm.at[pl.ds(base + boff, BLK)], bblk)
            pltpu.sync_copy(t_hbm.at[pl.ds(base + boff, BLK)], tblk)
            pltpu.sync_copy(y_hbm.at[pl.ds(base + boff, BLK)], yblk)
            pltpu.sync_copy(x_hbm.at[pl.ds(base + boff, BLK)], xblk)

            def key_step(v, carry2):
                off = v * 16
                bv = bblk[pl.ds(off, 16)]
                tv = tblk[pl.ds(off, 16)]
                yv = yblk[pl.ds(off, 16)]
                xv = xblk[pl.ds(off, 16)]
                grp = jnp.where(tv >= 5, one, zero) + jnp.where(tv >= 10, one, zero)
                kb[pl.ds(boff + off, 16)] = (bv * HW + yv * W + xv) * 3 + grp
                return carry2

            lax.fori_loop(0, BLK // 16, key_step, jnp.int32(0))
            return carry

        lax.fori_loop(0, SH // BLK, key_block, jnp.int32(0))

        zv = jnp.zeros((16,), jnp.float32)
        for r in range(ZR):
            for c16 in range(128 // 16):
                zbuf[r, pl.ds(c16 * 16, 16)] = zv

        for j in range(PCC):
            lo3 = (c * PCC + j) * (CHUNK_ROWS * 3)
            hi3 = lo3 + CHUNK_ROWS * 3

            for z in range(NZ):
                pltpu.sync_copy(zbuf, acc.at[pl.ds(s * STRIPE + z * ZR, ZR)])
            plsc.subcore_barrier()

            def scan_step(v, fill):
                off = v * 16
                kv = kb[pl.ds(off, 16)]
                m = (kv >= lo3) & (kv < hi3)
                pkv = ((base + off + iota) << SHIFT) | (kv - lo3)
                plsc.store_compressed(pk.at[pl.ds(fill, 16)], pkv, mask=m)
                return fill + plsc.all_reduce_population_count(m)[0]

            total = lax.fori_loop(0, SH // 16, scan_step, jnp.int32(0))

            for u in range(KB // 16):
                pk[pl.ds(total + u * 16, 16)] = jnp.full((16,), DUMMY, jnp.int32)

            nb = (total + KB - 1) // KB
            nb2 = (nb + 1) // 2

            def stage(b, gst, lst):
                for u in range(KB // 16):
                    pkv = pk[pl.ds(b * KB + u * 16, 16)]
                    gv = lax.shift_right_logical(pkv, SHIFT)
                    lv = pkv & ((1 << SHIFT) - 1)
                    for cc in range(CF):
                        gst[cc, pl.ds(u * 16, 16)] = gv * CF + cc
                        lst[cc, pl.ds(u * 16, 16)] = lv * CF + cc

            def fire(gst, rows, sem):
                for cc in range(CF):
                    pltpu.async_copy(feat_hbm.at[gst.at[cc]],
                                     rows.at[pl.ds(cc * KB, KB)], sem)

            def drain_scatter(gst, lst, rows, sem):
                for cc in range(CF):
                    pltpu.make_async_copy(feat_hbm.at[gst.at[cc]],
                                          rows.at[pl.ds(cc * KB, KB)],
                                          sem).wait()
                for cc in range(CF):
                    pltpu.sync_copy(rows.at[pl.ds(cc * KB, KB)],
                                    acc.at[lst.at[cc]], add=True)

            @pl.when(nb > 0)
            def _prologue():
                stage(0, gstA, lstA)
                fire(gstA, rowsA, semA)

            def pair_step(p, carry):
                b1 = 2 * p + 1

                @pl.when(b1 < nb)
                def _fire_b():
                    stage(b1, gstB, lstB)
                    fire(gstB, rowsB, semB)

                drain_scatter(gstA, lstA, rowsA, semA)

                @pl.when(b1 < nb)
                def _b_side():
                    @pl.when(b1 + 1 < nb)
                    def _fire_a():
                        stage(b1 + 1, gstA, lstA)
                        fire(gstA, rowsA, semA)

                    drain_scatter(gstB, lstB, rowsB, semB)

                return carry

            lax.fori_loop(0, nb2, pair_step, jnp.int32(0))
            plsc.subcore_barrier()

            pltpu.sync_copy(acc.at[pl.ds(s * STRIPE, STRIPE)],
                            g_hbm.at[pl.ds(lo3 * CF + s * STRIPE, STRIPE)])

    return sc_scatter, NPAD, P


def _make_tc_matmul(HW, K3, O, R):
    grid = (B, HW // R)

    def body(g_ref, w_ref, bias_ref, o_ref):
        o_ref[0] = lax.dot_general(
            w_ref[...], g_ref[0],
            dimension_numbers=(((1,), (1,)), ((), ())),
            preferred_element_type=jnp.float32,
            precision=lax.Precision.HIGHEST,
        ) + bias_ref[...]

    return pl.pallas_call(
        body,
        grid=grid,
        in_specs=[
            pl.BlockSpec((1, R, K3), lambda b, r: (b, r, 0)),
            pl.BlockSpec((O, K3), lambda b, r: (0, 0)),
            pl.BlockSpec((O, 1), lambda b, r: (0, 0)),
        ],
        out_specs=pl.BlockSpec((1, O, R), lambda b, r: (b, 0, r)),
        out_shape=jax.ShapeDtypeStruct((B, O, HW), jnp.float32),
    )


_SC_KERNELS = []
_TC_KERNELS = []
for _C, _H, _W, _N, _CR, _KB, _BLK in SCALES:
    _SC_KERNELS.append(_make_sc_scatter(_C, _H, _W, _N, _CR, _KB, _BLK))
    _TC_KERNELS.append(_make_tc_matmul(_H * _W, 3 * _C, _C, min(_H * _W, 2048)))


def _one_scale(i, feat, idx, Wm, bvec):
    C, H, W, N, _, _, _ = SCALES[i]
    sc_scatter, NPAD, P = _SC_KERNELS[i]
    pad = NPAD - N
    bcol = jnp.concatenate([idx[:, 0], jnp.full((pad,), B, jnp.int32)])
    tcol = jnp.concatenate([idx[:, 1], jnp.zeros((pad,), jnp.int32)])
    ycol = jnp.concatenate([idx[:, 2], jnp.zeros((pad,), jnp.int32)])
    xcol = jnp.concatenate([idx[:, 3], jnp.zeros((pad,), jnp.int32)])
    G = sc_scatter(bcol, tcol, ycol, xcol, feat.reshape(N * (C // 128), 128))
    G3 = G.reshape(B, H * W, 3 * C)
    out = _TC_KERNELS[i](G3, Wm, bvec[:, None])
    return out.reshape(B, C, H, W)


def kernel(feat0, idx0, feat1, idx1, feat2, idx2, W0, b0, W1, b1, W2, b2,
           batch_size):
    o0 = _one_scale(0, feat0, idx0, W0, b0)
    o1 = _one_scale(1, feat1, idx1, W1, b1)
    o2 = _one_scale(2, feat2, idx2, W2, b2)
    return (o0, o1, o2)


# speedup vs baseline: 2.3546x; 1.3090x over previous
import functools

import jax
import jax.numpy as jnp
from jax import lax
from jax.experimental import pallas as pl
from jax.experimental.pallas import tpu as pltpu
from jax.experimental.pallas import tpu_sc as plsc

B = 2
NSUB = 16
NCORE = 2
SHIFT = 14

SCALES = [
    (128, 128, 128, 200000, 4096, 16, 256),
    (256, 64, 64, 100000, 2048, 16, 512),
    (512, 32, 32, 50000, 1024, 16, 256),
]
ZR = 8


def _make_sc_scatter(C, H, W, N, CHUNK_ROWS, KB, BLK):
    HW = H * W
    P = B * HW
    CF = C // 128
    NCHUNK = P // CHUNK_ROWS
    PCC = NCHUNK // NCORE
    SH = ((N + NSUB - 1) // NSUB + BLK - 1) // BLK * BLK
    NPAD = SH * NSUB
    ACC_ROWS = (CHUNK_ROWS * 3 + 8) * CF
    DUMMY = CHUNK_ROWS * 3
    STRIPE = (CHUNK_ROWS * 3 // NSUB) * CF
    NZ = STRIPE // ZR
    assert DUMMY < (1 << SHIFT) and (NPAD << SHIFT) < (1 << 32)

    mesh = plsc.VectorSubcoreMesh(core_axis_name="c", subcore_axis_name="s")

    @functools.partial(
        pl.kernel,
        mesh=mesh,
        out_type=jax.ShapeDtypeStruct((P * 3 * CF, 128), jnp.float32),
        compiler_params=pltpu.CompilerParams(needs_layout_passes=False),
        scratch_types=[
            pltpu.VMEM((SH,), jnp.int32),
            pltpu.VMEM((SH + KB,), jnp.int32),
            pltpu.VMEM((BLK,), jnp.int32),
            pltpu.VMEM((BLK,), jnp.int32),
            pltpu.VMEM((BLK,), jnp.int32),
            pltpu.VMEM((BLK,), jnp.int32),
            pltpu.VMEM((CF * KB, 128), jnp.float32),
            pltpu.VMEM((CF * KB, 128), jnp.float32),
            pltpu.VMEM((CF, KB), jnp.int32),
            pltpu.VMEM((CF, KB), jnp.int32),
            pltpu.VMEM((CF, KB), jnp.int32),
            pltpu.VMEM((CF, KB), jnp.int32),
            pltpu.VMEM((ZR, 128), jnp.float32),
            pltpu.VMEM_SHARED((ACC_ROWS, 128), jnp.float32),
            pltpu.SemaphoreType.DMA,
            pltpu.SemaphoreType.DMA,
        ],
    )
    def sc_scatter(b_hbm, t_hbm, y_hbm, x_hbm, feat_hbm, g_hbm,
                   kb, pk, bblk, tblk, yblk, xblk, rowsA, rowsB, gstA, gstB,
                   lstA, lstB, zbuf, acc, semA, semB):
        c = lax.axis_index("c")
        s = lax.axis_index("s")
        base = s * SH
        iota = lax.iota(jnp.int32, 16)
        one = jnp.full((16,), 1, jnp.int32)
        zero = jnp.zeros((16,), jnp.int32)

        def key_block(blk, carry):
            boff = blk * BLK
            pltpu.sync_copy(b_hbm.at[pl.ds(base + boff, BLK)], bblk)
            pltpu.sync_copy(t_hbm.at[pl.ds(base + boff, BLK)], tblk)
            pltpu.sync_copy(y_hbm.at[pl.ds(base + boff, BLK)], yblk)
            pltpu.sync_copy(x_hbm.at[pl.ds(base + boff, BLK)], xblk)

            def key_step(v, carry2):
                off = v * 16
                bv = bblk[pl.ds(off, 16)]
                tv = tblk[pl.ds(off, 16)]
                yv = yblk[pl.ds(off, 16)]
                xv = xblk[pl.ds(off, 16)]
                grp = jnp.where(tv >= 5, one, zero) + jnp.where(tv >= 10, one, zero)
                kb[pl.ds(boff + off, 16)] = (bv * HW + yv * W + xv) * 3 + grp
                return carry2

            lax.fori_loop(0, BLK // 16, key_step, jnp.int32(0))
            return carry

        lax.fori_loop(0, SH // BLK, key_block, jnp.int32(0))

        zv = jnp.zeros((16,), jnp.float32)
        for r in range(ZR):
            for c16 in range(128 // 16):
                zbuf[r, pl.ds(c16 * 16, 16)] = zv

        for j in range(PCC):
            lo3 = (c * PCC + j) * (CHUNK_ROWS * 3)
            hi3 = lo3 + CHUNK_ROWS * 3

            for z in range(NZ):
                pltpu.sync_copy(zbuf, acc.at[pl.ds(s * STRIPE + z * ZR, ZR)])
            plsc.subcore_barrier()

            def scan_step(v, fill):
                off = v * 16
                kv = kb[pl.ds(off, 16)]
                m = (kv >= lo3) & (kv < hi3)
                pkv = ((base + off + iota) << SHIFT) | (kv - lo3)
                plsc.store_compressed(pk.at[pl.ds(fill, 16)], pkv, mask=m)
                return fill + plsc.all_reduce_population_count(m)[0]

            total = lax.fori_loop(0, SH // 16, scan_step, jnp.int32(0))

            for u in range(KB // 16):
                pk[pl.ds(total + u * 16, 16)] = jnp.full((16,), DUMMY, jnp.int32)

            nb = (total + KB - 1) // KB
            nb2 = (nb + 1) // 2

            def stage(b, gst, lst):
                for u in range(KB // 16):
                    pkv = pk[pl.ds(b * KB + u * 16, 16)]
                    gv = lax.shift_right_logical(pkv, SHIFT)
                    lv = pkv & ((1 << SHIFT) - 1)
                    for cc in range(CF):
                        gst[cc, pl.ds(u * 16, 16)] = gv * CF + cc
                        lst[cc, pl.ds(u * 16, 16)] = lv * CF + cc

            def fire(gst, rows, sem):
                for cc in range(CF):
                    pltpu.async_copy(feat_hbm.at[gst.at[cc]],
                                     rows.at[pl.ds(cc * KB, KB)], sem)

            def drain_scatter(gst, lst, rows, sem):
                for cc in range(CF):
                    pltpu.make_async_copy(feat_hbm.at[gst.at[cc]],
                                          rows.at[pl.ds(cc * KB, KB)],
                                          sem).wait()
                for cc in range(CF):
                    pltpu.sync_copy(rows.at[pl.ds(cc * KB, KB)],
                                    acc.at[lst.at[cc]], add=True)

            @pl.when(nb > 0)
            def _prologue():
                stage(0, gstA, lstA)
                fire(gstA, rowsA, semA)

            def pair_step(p, carry):
                b1 = 2 * p + 1

                @pl.when(b1 < nb)
                def _fire_b():
                    stage(b1, gstB, lstB)
                    fire(gstB, rowsB, semB)

                drain_scatter(gstA, lstA, rowsA, semA)

                @pl.when(b1 < nb)
                def _b_side():
                    @pl.when(b1 + 1 < nb)
                    def _fire_a():
                        stage(b1 + 1, gstA, lstA)
                        fire(gstA, rowsA, semA)

                    drain_scatter(gstB, lstB, rowsB, semB)

                return carry

            lax.fori_loop(0, nb2, pair_step, jnp.int32(0))
            plsc.subcore_barrier()

            pltpu.sync_copy(acc.at[pl.ds(s * STRIPE, STRIPE)],
                            g_hbm.at[pl.ds(lo3 * CF + s * STRIPE, STRIPE)])

    return sc_scatter, NPAD, P


def _make_tc_matmul(HW, K3, O, R):
    grid = (B, HW // R)

    def body(g_ref, w_ref, bias_ref, o_ref):
        o_ref[0] = lax.dot_general(
            w_ref[...], g_ref[0],
            dimension_numbers=(((1,), (1,)), ((), ())),
            preferred_element_type=jnp.float32,
            precision=lax.Precision.HIGHEST,
        ) + bias_ref[...]

    return pl.pallas_call(
        body,
        grid=grid,
        in_specs=[
            pl.BlockSpec((1, R, K3), lambda b, r: (b, r, 0)),
            pl.BlockSpec((O, K3), lambda b, r: (0, 0)),
            pl.BlockSpec((O, 1), lambda b, r: (0, 0)),
        ],
        out_specs=pl.BlockSpec((1, O, R), lambda b, r: (b, 0, r)),
        out_shape=jax.ShapeDtypeStruct((B, O, HW), jnp.float32),
    )


_SC_KERNELS = []
_TC_KERNELS = []
for _C, _H, _W, _N, _CR, _KB, _BLK in SCALES:
    _SC_KERNELS.append(_make_sc_scatter(_C, _H, _W, _N, _CR, _KB, _BLK))
    _TC_KERNELS.append(_make_tc_matmul(_H * _W, 3 * _C, _C, min(_H * _W, 2048)))


def _one_scale(i, feat, idx, Wm, bvec):
    C, H, W, N, _, _, _ = SCALES[i]
    sc_scatter, NPAD, P = _SC_KERNELS[i]
    pad = NPAD - N
    bcol = jnp.concatenate([idx[:, 0], jnp.full((pad,), B, jnp.int32)])
    tcol = jnp.concatenate([idx[:, 1], jnp.zeros((pad,), jnp.int32)])
    ycol = jnp.concatenate([idx[:, 2], jnp.zeros((pad,), jnp.int32)])
    xcol = jnp.concatenate([idx[:, 3], jnp.zeros((pad,), jnp.int32)])
    G = sc_scatter(bcol, tcol, ycol, xcol, feat.reshape(N * (C // 128), 128))
    G3 = G.reshape(B, H * W, 3 * C)
    out = _TC_KERNELS[i](G3, Wm, bvec[:, None])
    return out.reshape(B, C, H, W)


def kernel(feat0, idx0, feat1, idx1, feat2, idx2, W0, b0, W1, b1, W2, b2,
           batch_size):
    o0 = _one_scale(0, feat0, idx0, W0, b0)
    o1 = _one_scale(1, feat1, idx1, W1, b1)
    o2 = _one_scale(2, feat2, idx2, W2, b2)
    return (o0, o1, o2)
